# baseline (speedup 1.0000x reference)
import functools

import jax
import jax.numpy as jnp
from jax import lax
from jax.experimental import pallas as pl
from jax.experimental.pallas import tpu as pltpu
from jax.experimental.pallas import tpu_sc as plsc

B = 16
S = 4096
LANES = 16
NQ = 4
QUARTER = S // NQ
QCHUNKS = QUARTER // LANES


def _make_sc_kernel():
  mesh = plsc.VectorSubcoreMesh(
      core_axis_name="c", subcore_axis_name="s", num_cores=1)

  @functools.partial(
      pl.kernel,
      mesh=mesh,
      out_type=jax.ShapeDtypeStruct((B, S), jnp.float32),
      scratch_types=[
          pltpu.VMEM((S,), jnp.float32),
          pltpu.SemaphoreType.DMA,
          pltpu.SemaphoreType.DMA,
      ],
      compiler_params=pltpu.CompilerParams(needs_layout_passes=False),
  )
  def cumsum_kernel(pad_hbm, out_hbm, buf, sem_in, sem_out):
    row = lax.axis_index("s")

    cps_in = [
        pltpu.async_copy(
            pad_hbm.at[row, pl.ds(q * QUARTER, QUARTER)],
            buf.at[pl.ds(q * QUARTER, QUARTER)],
            sem_in,
        )
        for q in range(NQ)
    ]

    lane15 = jnp.full((LANES,), LANES - 1, jnp.int32)

    def scan_body(i, carry):
      base = i * LANES
      v = buf[pl.ds(base, LANES)]
      buf[pl.ds(base, LANES)] = plsc.cumsum(v) + carry
      return plsc.load_gather(buf, [lane15 + base])

    cps_out = []
    carry = jnp.full((LANES,), -1.0, jnp.float32)
    for q in range(NQ):
      cps_in[q].wait()
      carry = lax.fori_loop(q * QCHUNKS, (q + 1) * QCHUNKS, scan_body, carry,
                            unroll=4)
      cps_out.append(
          pltpu.async_copy(
              buf.at[pl.ds(q * QUARTER, QUARTER)],
              out_hbm.at[row, pl.ds(q * QUARTER, QUARTER)],
              sem_out,
          ))
    for cp in cps_out:
      cp.wait()

  return cumsum_kernel


_sc_cumsum = _make_sc_kernel()


@jax.jit
def kernel(pad_masks):
  return _sc_cumsum(pad_masks)
